# Initial kernel scaffold; baseline (speedup 1.0000x reference)
import jax
import jax.numpy as jnp
from jax.experimental import pallas as pl


def kernel(feature, adj, shuf, A, I, sparse, epoch, msk, samp_bias1, samp_bias2, W, b, Z, U, Wd):
    raise NotImplementedError("write your pallas kernel here")


# speedup vs baseline: 1.4308x; 1.4308x over previous
import functools
import math

import jax
import jax.numpy as jnp
from jax.experimental import pallas as pl
from jax.experimental.pallas import tpu as pltpu

_G = 2
_N = 4096
_FT = 512
_HID = 128
_CLUS = 32
_BN = 256
_NT = _N // _BN


def _pre_body(f_ref, s_ref, w_ref, b_ref, hcat_ref):
    w = w_ref[0]
    bb = b_ref[0, 0]
    h1 = jnp.dot(f_ref[0, 0], w, preferred_element_type=jnp.float32) + bb[None, :]
    h2 = jnp.dot(s_ref[0, 0], w, preferred_element_type=jnp.float32) + bb[None, :]
    hcat_ref[0] = jnp.concatenate([h1, h2], axis=1)


def _main_body(adj_ref, hcat_ref, z_ref, wd_ref, b1_ref, b2_ref,
               u1_ref, h_ref, sc1_ref, sc2_ref, osum_ref):
    a = adj_ref[0, 0]
    hc = hcat_ref[0]
    u = jnp.dot(a, hc, preferred_element_type=jnp.float32)
    u = jnp.maximum(u, 0.0)
    u1 = u[:, :_HID]
    u2 = u[:, _HID:]
    z = z_ref[0, 0]
    scores = jax.lax.dot_general(u1, z, (((1,), (1,)), ((), ())),
                                 preferred_element_type=jnp.float32)
    scores = scores * (1.0 / math.sqrt(float(_HID)))
    m = jnp.max(scores, axis=1, keepdims=True)
    e = jnp.exp(scores - m)
    h = e / jnp.sum(e, axis=1, keepdims=True)
    s = jnp.dot(h, z, preferred_element_type=jnp.float32)
    swd = jnp.dot(s, wd_ref[...], preferred_element_type=jnp.float32)
    sc1_ref[0, 0] = jnp.sum(swd * u1, axis=1) + b1_ref[0]
    sc2_ref[0, 0] = jnp.sum(swd * u2, axis=1) + b2_ref[0]
    u1_ref[0] = u1
    h_ref[0] = h
    cl = jnp.sum(h * h, axis=1)
    part = -jnp.sum(jax.nn.log_sigmoid(cl)) / float(_N)
    first = (pl.program_id(0) == 0) & (pl.program_id(1) == 0)

    @pl.when(first)
    def _():
        osum_ref[0, 0] = part

    @pl.when(jnp.logical_not(first))
    def _():
        osum_ref[0, 0] += part


def _apass_body(a_ref, ht_ref, u1_ref, uu_ref,
                lsum_ref, rsum_ref, m1_ref):
    n = pl.program_id(0)
    base = n * _BN
    a = a_ref[0]
    d2 = jnp.sum(a, axis=1, keepdims=True)
    rows = jax.lax.broadcasted_iota(jnp.int32, (_BN, _N), 0) + base
    cols = jax.lax.broadcasted_iota(jnp.int32, (_BN, _N), 1)
    txt = jnp.where(rows == cols, d2 - a, -a).astype(jnp.bfloat16)
    htt = ht_ref[:, pl.ds(base, _BN)]
    contrib = jnp.dot(htt, txt, preferred_element_type=jnp.float32)
    u1b = u1_ref[...]
    comb = (u1b[0] + u1b[1]) * 0.5
    rpart = jnp.sum((uu_ref[0] - comb) ** 2)
    first = n == 0

    @pl.when(first)
    def _():
        rsum_ref[0, 0] = rpart
        m1_ref[...] = contrib

    @pl.when(jnp.logical_not(first))
    def _():
        rsum_ref[0, 0] += rpart
        m1_ref[...] += contrib

    @pl.when(n == _NT - 1)
    def _():
        m1q = m1_ref[...].astype(jnp.bfloat16).astype(jnp.float32)
        hf = ht_ref[...].astype(jnp.float32)
        lsum_ref[0, 0] = jnp.sum(m1q * hf)


@jax.jit
def _run(feature, adj, shuf, A, samp_bias1, samp_bias2, W, b, Z, U, Wd):
    f32 = jnp.float32
    hcat = pl.pallas_call(
        _pre_body,
        grid=(_G, _NT),
        in_specs=[
            pl.BlockSpec((1, 1, _BN, _FT), lambda g, n: (g, 0, n, 0)),
            pl.BlockSpec((1, 1, _BN, _FT), lambda g, n: (g, 0, n, 0)),
            pl.BlockSpec((1, _FT, _HID), lambda g, n: (g, 0, 0)),
            pl.BlockSpec((1, 1, _HID), lambda g, n: (g, 0, 0)),
        ],
        out_specs=pl.BlockSpec((1, _BN, 2 * _HID), lambda g, n: (g, n, 0)),
        out_shape=jax.ShapeDtypeStruct((_G, _N, 2 * _HID), f32),
    )(feature, shuf, W, b[:, None, :])

    u1, h, sc1, sc2, osum = pl.pallas_call(
        _main_body,
        grid=(_G, _NT),
        in_specs=[
            pl.BlockSpec((1, 1, _BN, _N), lambda g, n: (g, 0, n, 0)),
            pl.BlockSpec((1, _N, 2 * _HID), lambda g, n: (g, 0, 0)),
            pl.BlockSpec((1, 1, _CLUS, _HID), lambda g, n: (g, 0, 0, 0)),
            pl.BlockSpec((_HID, _HID), lambda g, n: (0, 0)),
            pl.BlockSpec((1, _BN), lambda g, n: (0, n)),
            pl.BlockSpec((1, _BN), lambda g, n: (0, n)),
        ],
        out_specs=[
            pl.BlockSpec((1, _BN, _HID), lambda g, n: (g, n, 0)),
            pl.BlockSpec((1, _BN, _CLUS), lambda g, n: (g, n, 0)),
            pl.BlockSpec((1, 1, _BN), lambda g, n: (g, 0, n)),
            pl.BlockSpec((1, 1, _BN), lambda g, n: (g, 0, n)),
            pl.BlockSpec((1, 1), lambda g, n: (0, 0), memory_space=pltpu.SMEM),
        ],
        out_shape=[
            jax.ShapeDtypeStruct((_G, _N, _HID), f32),
            jax.ShapeDtypeStruct((_G, _N, _CLUS), f32),
            jax.ShapeDtypeStruct((_G, 1, _N), f32),
            jax.ShapeDtypeStruct((_G, 1, _N), f32),
            jax.ShapeDtypeStruct((1, 1), f32),
        ],
    )(adj, hcat, Z, Wd, samp_bias1, samp_bias2)

    hallt = jnp.concatenate([h[0], h[1]], axis=1).T.astype(jnp.bfloat16)
    lsum, rsum = pl.pallas_call(
        _apass_body,
        grid=(_NT,),
        in_specs=[
            pl.BlockSpec((1, _BN, _N), lambda n: (0, n, 0)),
            pl.BlockSpec((2 * _CLUS, _N), lambda n: (0, 0)),
            pl.BlockSpec((_G, _BN, _HID), lambda n: (0, n, 0)),
            pl.BlockSpec((1, _BN, _HID), lambda n: (0, n, 0)),
        ],
        out_specs=[
            pl.BlockSpec((1, 1), lambda n: (0, 0), memory_space=pltpu.SMEM),
            pl.BlockSpec((1, 1), lambda n: (0, 0), memory_space=pltpu.SMEM),
        ],
        out_shape=[
            jax.ShapeDtypeStruct((1, 1), f32),
            jax.ShapeDtypeStruct((1, 1), f32),
        ],
        scratch_shapes=[
            pltpu.VMEM((2 * _CLUS, _N), f32),
        ],
    )(A, hallt, u1, U)

    logits_all = jnp.concatenate([sc1, sc2], axis=2)
    return logits_all, lsum[0, 0], osum[0, 0], rsum[0, 0]


def kernel(feature, adj, shuf, A, I, sparse, epoch, msk, samp_bias1,
           samp_bias2, W, b, Z, U, Wd):
    return _run(feature, adj, shuf, A, samp_bias1, samp_bias2, W, b, Z, U, Wd)
